# baseline (speedup 1.0000x reference)
import functools

import jax
import jax.numpy as jnp
from jax import lax
from jax.experimental import pallas as pl
from jax.experimental.pallas import tpu as pltpu
from jax.experimental.pallas import tpu_sc as plsc

D = 128
NW = 32
CHUNK = 256
NBUF = 3
UNROLL = 4
STAGE_W = 104
OUT_OFF = 64
OUT_W = STAGE_W - OUT_OFF


def _decode_token(buf, t):
    iota = lax.iota(jnp.int32, 16)
    line0 = buf[t, pl.ds(0, 16)]
    act = plsc.all_reduce_population_count((line0 >= 0.5) & (iota < 2))
    mask = act == 2

    def field_argmax(off):
        v = buf[t, pl.ds(off, 16)]
        return plsc.all_reduce_ffs(v == jnp.max(v))

    a_lo = field_argmax(2)
    a_hi = field_argmax(18)
    b_lo = field_argmax(34)
    b_hi = field_argmax(50)
    a_val = a_lo + (a_hi << 4)
    b_val = b_lo + (b_hi << 4)
    r = (a_val * b_val) & 255
    r_lo = r & 15
    r_hi = r >> 4
    zero = jnp.float32(0.0)
    two = jnp.float32(2.0)
    lo = buf[t, pl.ds(66, 16)]
    buf[t, pl.ds(66, 16)] = lo + jnp.where((iota == r_lo) & mask, two, zero)
    hi = buf[t, pl.ds(82, 16)]
    buf[t, pl.ds(82, 16)] = hi + jnp.where((iota == r_hi) & mask, two, zero)


def _make_sc_kernel(n_tokens):
    tpw = n_tokens // NW
    n_chunks = tpw // CHUNK
    mesh = plsc.VectorSubcoreMesh(core_axis_name="c", subcore_axis_name="s")

    @functools.partial(
        pl.kernel,
        mesh=mesh,
        out_type=jax.ShapeDtypeStruct((n_tokens, D), jnp.float32),
        scratch_types=(
            [pltpu.VMEM((CHUNK, STAGE_W), jnp.float32)] * NBUF
            + [pltpu.SemaphoreType.DMA] * (2 * NBUF + 2)
        ),
        compiler_params=pltpu.CompilerParams(
            needs_layout_passes=False, use_tc_tiling_on_sc=False),
    )
    def k(x_hbm, out_hbm, *scratch):
        bufs = scratch[:NBUF]
        in_sems = scratch[NBUF:2 * NBUF]
        out_sems = scratch[2 * NBUF:2 * NBUF + NBUF]
        cp_sems = scratch[2 * NBUF + NBUF:]
        wid = lax.axis_index("s") * 2 + lax.axis_index("c")
        w_base = wid * tpw

        cp0 = pltpu.async_copy(
            x_hbm.at[pl.ds(w_base, tpw), pl.ds(0, OUT_OFF)],
            out_hbm.at[pl.ds(w_base, tpw), pl.ds(0, OUT_OFF)],
            cp_sems[0])
        cp1 = pltpu.async_copy(
            x_hbm.at[pl.ds(w_base, tpw), pl.ds(STAGE_W, D - STAGE_W)],
            out_hbm.at[pl.ds(w_base, tpw), pl.ds(STAGE_W, D - STAGE_W)],
            cp_sems[1])

        def start_in(c):
            tok0 = w_base + c * CHUNK
            return pltpu.async_copy(
                x_hbm.at[pl.ds(tok0, CHUNK), pl.ds(0, STAGE_W)],
                bufs[c % NBUF], in_sems[c % NBUF])

        def start_out(c):
            tok0 = w_base + c * CHUNK
            return pltpu.async_copy(
                bufs[c % NBUF].at[:, pl.ds(OUT_OFF, OUT_W)],
                out_hbm.at[pl.ds(tok0, CHUNK), pl.ds(OUT_OFF, OUT_W)],
                out_sems[c % NBUF])

        ins = {c: start_in(c) for c in range(min(2, n_chunks))}
        outs = {}
        for c in range(n_chunks):
            buf = bufs[c % NBUF]
            ins.pop(c).wait()

            def tok_body(i, carry2, buf=buf):
                for u in range(UNROLL):
                    _decode_token(buf, i * UNROLL + u)
                return carry2

            lax.fori_loop(0, CHUNK // UNROLL, tok_body, 0)
            outs[c] = start_out(c)
            nxt = c + 2
            if nxt < n_chunks:
                if nxt - NBUF >= 0:
                    outs.pop(nxt - NBUF).wait()
                ins[nxt] = start_in(nxt)
        for c in sorted(outs):
            outs.pop(c).wait()
        cp0.wait()
        cp1.wait()

    return k


@jax.jit
def kernel(x_bd, mul_table):
    del mul_table
    b, s, d = x_bd.shape
    n = b * s
    out = _make_sc_kernel(n)(x_bd.reshape(n, d))
    return out.reshape(b, s, d)


# speedup vs baseline: 20.3706x; 20.3706x over previous
import functools

import jax
import jax.numpy as jnp
from jax import lax
from jax.experimental import pallas as pl
from jax.experimental.pallas import tpu as pltpu
from jax.experimental.pallas import tpu_sc as plsc

D = 128
NW = 32
CHUNK = 128
NBUF = 4
UNROLL = 4


def _decode_token(buf, t):
    iota = lax.iota(jnp.int32, 16)
    line0 = buf[t, pl.ds(0, 16)]
    act = plsc.all_reduce_population_count((line0 >= 0.5) & (iota < 2))
    mask = act == 2

    def field_argmax(off):
        v = buf[t, pl.ds(off, 16)]
        return plsc.all_reduce_ffs(v == jnp.max(v))

    a_lo = field_argmax(2)
    a_hi = field_argmax(18)
    b_lo = field_argmax(34)
    b_hi = field_argmax(50)
    a_val = a_lo + (a_hi << 4)
    b_val = b_lo + (b_hi << 4)
    r = (a_val * b_val) & 255
    r_lo = r & 15
    r_hi = r >> 4
    zero = jnp.float32(0.0)
    two = jnp.float32(2.0)
    lo = buf[t, pl.ds(66, 16)]
    buf[t, pl.ds(66, 16)] = lo + jnp.where((iota == r_lo) & mask, two, zero)
    hi = buf[t, pl.ds(82, 16)]
    buf[t, pl.ds(82, 16)] = hi + jnp.where((iota == r_hi) & mask, two, zero)


def _make_sc_kernel(n_tokens):
    tpw = n_tokens // NW
    n_chunks = tpw // CHUNK
    mesh = plsc.VectorSubcoreMesh(core_axis_name="c", subcore_axis_name="s")

    @functools.partial(
        pl.kernel,
        mesh=mesh,
        out_type=jax.ShapeDtypeStruct((n_tokens, D), jnp.float32),
        scratch_types=(
            [pltpu.VMEM((CHUNK, D), jnp.float32)] * NBUF
            + [pltpu.SemaphoreType.DMA] * (2 * NBUF)
        ),
        compiler_params=pltpu.CompilerParams(
            needs_layout_passes=False, use_tc_tiling_on_sc=False),
    )
    def k(x_hbm, out_hbm, *scratch):
        bufs = scratch[:NBUF]
        in_sems = scratch[NBUF:2 * NBUF]
        out_sems = scratch[2 * NBUF:]
        wid = lax.axis_index("s") * 2 + lax.axis_index("c")
        w_base = wid * tpw

        def start_in(c):
            tok0 = w_base + c * CHUNK
            return pltpu.async_copy(
                x_hbm.at[pl.ds(tok0, CHUNK)], bufs[c % NBUF],
                in_sems[c % NBUF])

        def start_out(c):
            tok0 = w_base + c * CHUNK
            return pltpu.async_copy(
                bufs[c % NBUF], out_hbm.at[pl.ds(tok0, CHUNK)],
                out_sems[c % NBUF])

        pre = NBUF - 1
        ins = {c: start_in(c) for c in range(min(pre, n_chunks))}
        outs = {}
        for c in range(n_chunks):
            buf = bufs[c % NBUF]
            ins.pop(c).wait()

            def tok_body(i, carry2, buf=buf):
                for u in range(UNROLL):
                    _decode_token(buf, i * UNROLL + u)
                return carry2

            lax.fori_loop(0, CHUNK // UNROLL, tok_body, 0)
            outs[c] = start_out(c)
            nxt = c + pre
            if nxt < n_chunks:
                if nxt - NBUF >= 0:
                    outs.pop(nxt - NBUF).wait()
                ins[nxt] = start_in(nxt)
        for c in sorted(outs):
            outs.pop(c).wait()

    return k


@jax.jit
def kernel(x_bd, mul_table):
    del mul_table
    b, s, d = x_bd.shape
    n = b * s
    out = _make_sc_kernel(n)(x_bd.reshape(n, d))
    return out.reshape(b, s, d)


# speedup vs baseline: 21.3988x; 1.0505x over previous
import functools

import jax
import jax.numpy as jnp
from jax import lax
from jax.experimental import pallas as pl
from jax.experimental.pallas import tpu as pltpu
from jax.experimental.pallas import tpu_sc as plsc

D = 128
NW = 32
CHUNK = 256
NBUF = 3
UNROLL = 4


def _decode_token(buf, t):
    iota = lax.iota(jnp.int32, 16)
    line0 = buf[t, pl.ds(0, 16)]
    act = plsc.all_reduce_population_count((line0 >= 0.5) & (iota < 2))
    mask = act == 2

    def field_argmax(off):
        v = buf[t, pl.ds(off, 16)]
        return plsc.all_reduce_ffs(v == jnp.max(v))

    a_lo = field_argmax(2)
    a_hi = field_argmax(18)
    b_lo = field_argmax(34)
    b_hi = field_argmax(50)
    a_val = a_lo + (a_hi << 4)
    b_val = b_lo + (b_hi << 4)
    r = (a_val * b_val) & 255
    r_lo = r & 15
    r_hi = r >> 4
    zero = jnp.float32(0.0)
    two = jnp.float32(2.0)
    lo = buf[t, pl.ds(66, 16)]
    buf[t, pl.ds(66, 16)] = lo + jnp.where((iota == r_lo) & mask, two, zero)
    hi = buf[t, pl.ds(82, 16)]
    buf[t, pl.ds(82, 16)] = hi + jnp.where((iota == r_hi) & mask, two, zero)


def _make_sc_kernel(n_tokens):
    tpw = n_tokens // NW
    n_chunks = tpw // CHUNK
    mesh = plsc.VectorSubcoreMesh(core_axis_name="c", subcore_axis_name="s")

    @functools.partial(
        pl.kernel,
        mesh=mesh,
        out_type=jax.ShapeDtypeStruct((n_tokens, D), jnp.float32),
        scratch_types=(
            [pltpu.VMEM((CHUNK, D), jnp.float32)] * NBUF
            + [pltpu.SemaphoreType.DMA] * (2 * NBUF)
        ),
        compiler_params=pltpu.CompilerParams(
            needs_layout_passes=False, use_tc_tiling_on_sc=False),
    )
    def k(x_hbm, out_hbm, *scratch):
        bufs = scratch[:NBUF]
        in_sems = scratch[NBUF:2 * NBUF]
        out_sems = scratch[2 * NBUF:]
        wid = lax.axis_index("s") * 2 + lax.axis_index("c")
        w_base = wid * tpw

        def start_in(c):
            tok0 = w_base + c * CHUNK
            return pltpu.async_copy(
                x_hbm.at[pl.ds(tok0, CHUNK)], bufs[c % NBUF],
                in_sems[c % NBUF])

        def start_out(c):
            tok0 = w_base + c * CHUNK
            return pltpu.async_copy(
                bufs[c % NBUF], out_hbm.at[pl.ds(tok0, CHUNK)],
                out_sems[c % NBUF])

        ins = {c: start_in(c) for c in range(min(2, n_chunks))}
        outs = {}
        for c in range(n_chunks):
            buf = bufs[c % NBUF]
            ins.pop(c).wait()

            def tok_body(i, carry2, buf=buf):
                for u in range(UNROLL):
                    _decode_token(buf, i * UNROLL + u)
                return carry2

            lax.fori_loop(0, CHUNK // UNROLL, tok_body, 0)
            outs[c] = start_out(c)
            nxt = c + 2
            if nxt < n_chunks:
                if nxt - NBUF >= 0:
                    outs.pop(nxt - NBUF).wait()
                ins[nxt] = start_in(nxt)
        for c in sorted(outs):
            outs.pop(c).wait()

    return k


@jax.jit
def kernel(x_bd, mul_table):
    del mul_table
    b, s, d = x_bd.shape
    n = b * s
    out = _make_sc_kernel(n)(x_bd.reshape(n, d))
    return out.reshape(b, s, d)
